# baseline (speedup 1.0000x reference)
import functools
import math

import jax
import jax.numpy as jnp
from jax import lax
from jax.experimental import pallas as pl
from jax.experimental.pallas import tpu as pltpu
from jax.experimental.pallas import tpu_sc as plsc

MODEL_DIM = 64
VOCAB = 1000000
SCALE = math.sqrt(MODEL_DIM)

NUM_CORES = 2
NUM_SUBCORES = 16
NUM_WORKERS = NUM_CORES * NUM_SUBCORES
LANES = 16

TBLK = 128
FULL_BLK = VOCAB // TBLK
TAIL = VOCAB - FULL_BLK * TBLK
NBLK = FULL_BLK + 1

CHUNK = 512
SUBGATHER = 128

_MESH = dict(core_axis_name="c", subcore_axis_name="s",
             num_cores=NUM_CORES, num_subcores=NUM_SUBCORES)


TC_W = 512
N_PAIR = VOCAB // 2


def _make_format_kernel():
    grid = (VOCAB + TC_W - 1) // TC_W

    def body(tt_ref, out_ref):
        pi = lax.broadcasted_iota(jnp.int32, (MODEL_DIM, TBLK), 0)
        ki = lax.broadcasted_iota(jnp.int32, (MODEL_DIM, TBLK), 1)
        i_even = (ki == 2 * pi).astype(jnp.float32)
        i_odd = (ki == 2 * pi + 1).astype(jnp.float32)
        dn = (((1,), (1,)), ((), ()))
        for s in range(TC_W // TBLK):
            blk = tt_ref[:, s * TBLK:(s + 1) * TBLK]
            even = lax.dot_general(i_even, blk, dn,
                                   preferred_element_type=jnp.float32)
            odd = lax.dot_general(i_odd, blk, dn,
                                  preferred_element_type=jnp.float32)
            r0 = s * (TBLK // 2)
            out_ref[r0:r0 + TBLK // 2, 0:MODEL_DIM] = even
            out_ref[r0:r0 + TBLK // 2, MODEL_DIM:2 * MODEL_DIM] = odd

    return pl.pallas_call(
        body,
        grid=grid,
        in_specs=[pl.BlockSpec((MODEL_DIM, TC_W), lambda j: (0, j))],
        out_specs=pl.BlockSpec((TC_W // 2, 2 * MODEL_DIM), lambda j: (j, 0)),
        out_shape=jax.ShapeDtypeStruct((N_PAIR, 2 * MODEL_DIM), jnp.float32),
    )


def _make_gather_kernel(B: int):
    b_per_w = B // NUM_WORKERS
    steps = b_per_w // CHUNK
    n_sub = CHUNK // SUBGATHER

    @functools.partial(
        pl.kernel,
        out_type=jax.ShapeDtypeStruct((B, MODEL_DIM), jnp.float32),
        mesh=plsc.VectorSubcoreMesh(**_MESH),
        scratch_types=[
            pltpu.VMEM((CHUNK,), jnp.int32),
            pltpu.VMEM((CHUNK, MODEL_DIM), jnp.float32),
            pltpu.SemaphoreType.DMA,
        ],
        compiler_params=pltpu.CompilerParams(use_tc_tiling_on_sc=False),
    )
    def gat(idx_hbm, table_hbm, out_hbm, idx_v, rows_v, sem):
        wid = lax.axis_index("s") * NUM_CORES + lax.axis_index("c")
        base = wid * b_per_w

        def step(s, _):
            off = pl.multiple_of(base + s * CHUNK, CHUNK)
            pltpu.sync_copy(idx_hbm.at[pl.ds(off, CHUNK)], idx_v)
            copies = [
                pltpu.async_copy(
                    table_hbm.at[idx_v.at[pl.ds(q * SUBGATHER, SUBGATHER)]],
                    rows_v.at[pl.ds(q * SUBGATHER, SUBGATHER), :],
                    sem)
                for q in range(n_sub)
            ]
            for c in copies:
                c.wait()

            @plsc.parallel_loop(0, CHUNK, unroll=8)
            def scale_row(i):
                for g in range(MODEL_DIM // LANES):
                    sl = pl.ds(g * LANES, LANES)
                    rows_v[i, sl] = rows_v[i, sl] * SCALE
            pltpu.sync_copy(rows_v, out_hbm.at[pl.ds(off, CHUNK), :])
            return 0

        lax.fori_loop(0, steps, step, 0)

    return gat


def kernel(x, table):
    B = x.size
    idx = x.reshape(B).astype(jnp.int32)
    table_r = _make_format_kernel()(table.T).reshape(VOCAB, MODEL_DIM)
    out = _make_gather_kernel(B)(idx, table_r)
    return out.reshape(x.shape + (MODEL_DIM,))


# speedup vs baseline: 1.2128x; 1.2128x over previous
import functools
import math

import jax
import jax.numpy as jnp
from jax import lax
from jax.experimental import pallas as pl
from jax.experimental.pallas import tpu as pltpu
from jax.experimental.pallas import tpu_sc as plsc

MODEL_DIM = 64
VOCAB = 1000000
N_PAIR = VOCAB // 2
SCALE = math.sqrt(MODEL_DIM)

NUM_CORES = 2
NUM_SUBCORES = 16
NUM_WORKERS = NUM_CORES * NUM_SUBCORES
LANES = 16

TT = 8
IB = 128

_MESH = dict(core_axis_name="c", subcore_axis_name="s",
             num_cores=NUM_CORES, num_subcores=NUM_SUBCORES)


def _make_lookup_kernel(T: int, N: int):
    n_ib = N // IB
    units = (T // TT) * n_ib
    units_w = units // NUM_WORKERS

    @functools.partial(
        pl.kernel,
        out_type=jax.ShapeDtypeStruct((T, MODEL_DIM, N), jnp.float32),
        mesh=plsc.VectorSubcoreMesh(**_MESH),
        scratch_types=[
            pltpu.VMEM((TT, IB), jnp.int32),
            pltpu.VMEM((TT, IB), jnp.int32),
            pltpu.VMEM((IB, 2 * MODEL_DIM), jnp.float32),
            pltpu.VMEM((MODEL_DIM, IB), jnp.float32),
            pltpu.SemaphoreType.DMA,
        ],
        compiler_params=pltpu.CompilerParams(use_tc_tiling_on_sc=True,
                                             needs_layout_passes=False),
    )
    def lkp(xt_hbm, pairs_hbm, out_hbm, idxt_v, pidx_v, prows_v, trans_v,
            sem):
        wid = lax.axis_index("s") * NUM_CORES + lax.axis_index("c")
        lane = lax.iota(jnp.int32, LANES)

        def unit(k, _):
            u = wid + NUM_WORKERS * k
            tb = u // n_ib
            ib = u % n_ib
            pltpu.sync_copy(
                xt_hbm.at[pl.ds(tb * TT, TT), pl.ds(ib * IB, IB)], idxt_v)

            @plsc.parallel_loop(0, TT * (IB // LANES), unroll=8)
            def mkpidx(q):
                r = q // (IB // LANES)
                c0 = (q % (IB // LANES)) * LANES
                pidx_v[r, pl.ds(c0, LANES)] = (
                    idxt_v[r, pl.ds(c0, LANES)] >> 1)

            for t8 in range(TT):
                pltpu.async_copy(pairs_hbm.at[pidx_v.at[t8, :]],
                                 prows_v, sem).wait()
                for gi in range(IB // LANES):
                    idx16 = idxt_v[t8, pl.ds(gi * LANES, LANES)]
                    hv = (idx16 & 1) * MODEL_DIM
                    rowv = lane + gi * LANES

                    @plsc.parallel_loop(0, MODEL_DIM, unroll=8)
                    def col(c):
                        val = plsc.load_gather(prows_v, [rowv, hv + c])
                        trans_v[c, pl.ds(gi * LANES, LANES)] = val * SCALE

                pltpu.sync_copy(
                    trans_v,
                    out_hbm.at[tb * TT + t8, :, pl.ds(ib * IB, IB)])
            return 0

        lax.fori_loop(0, units_w, unit, 0)

    return lkp


def kernel(x, table):
    T, N = x.shape[1], x.shape[0]
    xt = x.T.astype(jnp.int32)
    pairs = table.reshape(N_PAIR, 2 * MODEL_DIM)
    out_t = _make_lookup_kernel(T, N)(xt, pairs)
    return jnp.transpose(out_t, (2, 0, 1))


# speedup vs baseline: 1.4822x; 1.2221x over previous
import functools
import math

import jax
import jax.numpy as jnp
from jax import lax
from jax.experimental import pallas as pl
from jax.experimental.pallas import tpu as pltpu
from jax.experimental.pallas import tpu_sc as plsc

MODEL_DIM = 64
VOCAB = 1000000
N_PAIR = VOCAB // 2
SCALE = math.sqrt(MODEL_DIM)

NUM_CORES = 2
NUM_SUBCORES = 16
NUM_WORKERS = NUM_CORES * NUM_SUBCORES
LANES = 16

TT = 8
IB = 128

_MESH = dict(core_axis_name="c", subcore_axis_name="s",
             num_cores=NUM_CORES, num_subcores=NUM_SUBCORES)


def _make_lookup_kernel(T: int, N: int):
    n_ib = N // IB
    units = (T // TT) * n_ib
    units_w = units // NUM_WORKERS

    @functools.partial(
        pl.kernel,
        out_type=jax.ShapeDtypeStruct((T, MODEL_DIM, N), jnp.float32),
        mesh=plsc.VectorSubcoreMesh(**_MESH),
        scratch_types=[
            pltpu.VMEM((TT, IB), jnp.int32),
            pltpu.VMEM((TT, IB), jnp.int32),
            pltpu.VMEM((TT, IB), jnp.int32),
            pltpu.VMEM((IB, 2 * MODEL_DIM), jnp.float32),
            pltpu.VMEM((IB, 2 * MODEL_DIM), jnp.float32),
            pltpu.VMEM((MODEL_DIM, IB), jnp.float32),
            pltpu.VMEM((MODEL_DIM, IB), jnp.float32),
            pltpu.SemaphoreType.DMA,
            pltpu.SemaphoreType.DMA,
        ],
        compiler_params=pltpu.CompilerParams(use_tc_tiling_on_sc=True,
                                             needs_layout_passes=False),
    )
    def lkp(xt_hbm, pairs_hbm, out_hbm, idxt_v, pidx_v, hv_v,
            prows_a, prows_b, trans_a, trans_b, gsem, wsem):
        wid = lax.axis_index("s") * NUM_CORES + lax.axis_index("c")
        lane = lax.iota(jnp.int32, LANES)
        prows = (prows_a, prows_b)
        trans = (trans_a, trans_b)

        def unit(k, _):
            u = wid + NUM_WORKERS * k
            tb = u // n_ib
            ib = u % n_ib
            pltpu.sync_copy(
                xt_hbm.at[pl.ds(tb * TT, TT), pl.ds(ib * IB, IB)], idxt_v)

            @plsc.parallel_loop(0, TT * (IB // LANES), unroll=8)
            def mkpidx(q):
                r = q // (IB // LANES)
                c0 = (q % (IB // LANES)) * LANES
                v = idxt_v[r, pl.ds(c0, LANES)]
                pidx_v[r, pl.ds(c0, LANES)] = v >> 1
                hv_v[r, pl.ds(c0, LANES)] = (v & 1) * MODEL_DIM

            gets = [None] * TT
            puts = [None] * TT
            gets[0] = pltpu.async_copy(pairs_hbm.at[pidx_v.at[0, :]],
                                       prows[0], gsem)
            for t8 in range(TT):
                cur = t8 & 1
                gets[t8].wait()
                if t8 + 1 < TT:
                    gets[t8 + 1] = pltpu.async_copy(
                        pairs_hbm.at[pidx_v.at[t8 + 1, :]],
                        prows[1 - cur], gsem)
                if t8 >= 2:
                    puts[t8 - 2].wait()
                src = prows[cur]
                dst = trans[cur]

                @plsc.parallel_loop(0, MODEL_DIM * (IB // LANES), unroll=8)
                def col(q):
                    gi = q // MODEL_DIM
                    c = q % MODEL_DIM
                    hv16 = hv_v[t8, pl.ds(gi * LANES, LANES)]
                    rowv = lane + gi * LANES
                    val = plsc.load_gather(src, [rowv, hv16 + c])
                    dst[c, pl.ds(gi * LANES, LANES)] = val * SCALE

                puts[t8] = pltpu.async_copy(
                    dst, out_hbm.at[tb * TT + t8, :, pl.ds(ib * IB, IB)],
                    wsem)
            puts[TT - 2].wait()
            puts[TT - 1].wait()
            return 0

        lax.fori_loop(0, units_w, unit, 0)

    return lkp


def kernel(x, table):
    T, N = x.shape[1], x.shape[0]
    xt = x.T.astype(jnp.int32)
    pairs = table.reshape(N_PAIR, 2 * MODEL_DIM)
    out_t = _make_lookup_kernel(T, N)(xt, pairs)
    return jnp.transpose(out_t, (2, 0, 1))


# speedup vs baseline: 1.4903x; 1.0055x over previous
import functools
import math

import jax
import jax.numpy as jnp
from jax import lax
from jax.experimental import pallas as pl
from jax.experimental.pallas import tpu as pltpu
from jax.experimental.pallas import tpu_sc as plsc

MODEL_DIM = 64
VOCAB = 1000000
N_PAIR = VOCAB // 2
SCALE = math.sqrt(MODEL_DIM)

NUM_CORES = 2
NUM_SUBCORES = 16
NUM_WORKERS = NUM_CORES * NUM_SUBCORES
LANES = 16

TT = 8
IB = 128

_MESH = dict(core_axis_name="c", subcore_axis_name="s",
             num_cores=NUM_CORES, num_subcores=NUM_SUBCORES)


def _make_lookup_kernel(T: int, N: int):
    n_ib = N // IB
    units = (T // TT) * n_ib
    units_w = units // NUM_WORKERS

    @functools.partial(
        pl.kernel,
        out_type=jax.ShapeDtypeStruct((T, MODEL_DIM, N), jnp.float32),
        mesh=plsc.VectorSubcoreMesh(**_MESH),
        scratch_types=[
            pltpu.VMEM((TT, IB), jnp.int32),
            pltpu.VMEM((TT, IB), jnp.int32),
            pltpu.VMEM((TT, IB), jnp.int32),
            pltpu.VMEM((IB, 2 * MODEL_DIM), jnp.float32),
            pltpu.VMEM((IB, 2 * MODEL_DIM), jnp.float32),
            pltpu.VMEM((MODEL_DIM, IB), jnp.float32),
            pltpu.VMEM((MODEL_DIM, IB), jnp.float32),
            pltpu.SemaphoreType.DMA,
            pltpu.SemaphoreType.DMA,
        ],
        compiler_params=pltpu.CompilerParams(use_tc_tiling_on_sc=True,
                                             needs_layout_passes=False),
    )
    def lkp(xt_hbm, pairs_hbm, out_hbm, idxt_v, pidx_v, hv_v,
            prows_a, prows_b, trans_a, trans_b, gsem, wsem):
        wid = lax.axis_index("s") * NUM_CORES + lax.axis_index("c")
        lane = lax.iota(jnp.int32, LANES)
        prows = (prows_a, prows_b)
        trans = (trans_a, trans_b)

        def unit(k, _):
            u = wid + NUM_WORKERS * k
            tb = u // n_ib
            ib = u % n_ib
            pltpu.sync_copy(
                xt_hbm.at[pl.ds(tb * TT, TT), pl.ds(ib * IB, IB)], idxt_v)

            @plsc.parallel_loop(0, TT * (IB // LANES), unroll=8)
            def mkpidx(q):
                r = q // (IB // LANES)
                c0 = (q % (IB // LANES)) * LANES
                v = idxt_v[r, pl.ds(c0, LANES)]
                pidx_v[r, pl.ds(c0, LANES)] = v >> 1
                hv_v[r, pl.ds(c0, LANES)] = (v & 1) * MODEL_DIM

            gets = [None] * TT
            puts = [None] * TT
            gets[0] = pltpu.async_copy(pairs_hbm.at[pidx_v.at[0, :]],
                                       prows[0], gsem)
            for t8 in range(TT):
                cur = t8 & 1
                gets[t8].wait()
                if t8 + 1 < TT:
                    gets[t8 + 1] = pltpu.async_copy(
                        pairs_hbm.at[pidx_v.at[t8 + 1, :]],
                        prows[1 - cur], gsem)
                if t8 >= 2:
                    puts[t8 - 2].wait()
                src = prows[cur]
                dst = trans[cur]

                for gi in range(IB // LANES):
                    hv16 = hv_v[t8, pl.ds(gi * LANES, LANES)]
                    rowv = lane + gi * LANES

                    @plsc.parallel_loop(0, MODEL_DIM, unroll=8)
                    def col(c):
                        val = plsc.load_gather(src, [rowv, hv16 + c])
                        dst[c, pl.ds(gi * LANES, LANES)] = val * SCALE

                puts[t8] = pltpu.async_copy(
                    dst, out_hbm.at[tb * TT + t8, :, pl.ds(ib * IB, IB)],
                    wsem)
            puts[TT - 2].wait()
            puts[TT - 1].wait()
            return 0

        lax.fori_loop(0, units_w, unit, 0)

    return lkp


def kernel(x, table):
    T, N = x.shape[1], x.shape[0]
    xt = x.T.astype(jnp.int32)
    pairs = table.reshape(N_PAIR, 2 * MODEL_DIM)
    out_t = _make_lookup_kernel(T, N)(xt, pairs)
    return jnp.transpose(out_t, (2, 0, 1))


# speedup vs baseline: 1.8784x; 1.2604x over previous
import functools
import math

import jax
import jax.numpy as jnp
from jax import lax
from jax.experimental import pallas as pl
from jax.experimental.pallas import tpu as pltpu
from jax.experimental.pallas import tpu_sc as plsc

MODEL_DIM = 64
VOCAB = 1000000
N_PAIR = VOCAB // 2
SCALE = math.sqrt(MODEL_DIM)

NUM_CORES = 2
NUM_SUBCORES = 16
NUM_WORKERS = NUM_CORES * NUM_SUBCORES
LANES = 16

TT = 8
IB = 128

_MESH = dict(core_axis_name="c", subcore_axis_name="s",
             num_cores=NUM_CORES, num_subcores=NUM_SUBCORES)

TC_W = 2048
N_GRID = 245
H = N_GRID * TC_W


def _make_pair_kernel():
    def body(lo_ref, hi_ref, out_ref):
        out_ref[:, 0:MODEL_DIM] = lo_ref[...].T
        out_ref[:, MODEL_DIM:2 * MODEL_DIM] = hi_ref[...].T

    return pl.pallas_call(
        body,
        grid=(N_GRID,),
        in_specs=[
            pl.BlockSpec((MODEL_DIM, TC_W), lambda j: (0, j)),
            pl.BlockSpec((MODEL_DIM, TC_W),
                         lambda j: (0, jnp.minimum(j + N_GRID,
                                                   VOCAB // TC_W))),
        ],
        out_specs=pl.BlockSpec((TC_W, 2 * MODEL_DIM), lambda j: (j, 0)),
        out_shape=jax.ShapeDtypeStruct((H, 2 * MODEL_DIM), jnp.float32),
    )


def _make_lookup_kernel(T: int, N: int):
    n_ib = N // IB
    units = (T // TT) * n_ib
    units_w = units // NUM_WORKERS

    @functools.partial(
        pl.kernel,
        out_type=jax.ShapeDtypeStruct((T, MODEL_DIM, N), jnp.float32),
        mesh=plsc.VectorSubcoreMesh(**_MESH),
        scratch_types=[
            pltpu.VMEM((TT, IB), jnp.int32),
            pltpu.VMEM((TT, IB), jnp.int32),
            pltpu.VMEM((TT, IB), jnp.int32),
            pltpu.VMEM((IB, 2 * MODEL_DIM), jnp.float32),
            pltpu.VMEM((IB, 2 * MODEL_DIM), jnp.float32),
            pltpu.VMEM((MODEL_DIM, IB), jnp.float32),
            pltpu.VMEM((MODEL_DIM, IB), jnp.float32),
            pltpu.SemaphoreType.DMA,
            pltpu.SemaphoreType.DMA,
        ],
        compiler_params=pltpu.CompilerParams(use_tc_tiling_on_sc=True,
                                             needs_layout_passes=False),
    )
    def lkp(xt_hbm, pairs_hbm, out_hbm, idxt_v, pidx_v, hv_v,
            prows_a, prows_b, trans_a, trans_b, gsem, wsem):
        wid = lax.axis_index("s") * NUM_CORES + lax.axis_index("c")
        lane = lax.iota(jnp.int32, LANES)
        prows = (prows_a, prows_b)
        trans = (trans_a, trans_b)

        def unit(k, _):
            u = wid + NUM_WORKERS * k
            tb = u // n_ib
            ib = u % n_ib
            pltpu.sync_copy(
                xt_hbm.at[pl.ds(tb * TT, TT), pl.ds(ib * IB, IB)], idxt_v)

            @plsc.parallel_loop(0, TT * (IB // LANES), unroll=8)
            def mkpidx(q):
                r = q // (IB // LANES)
                c0 = (q % (IB // LANES)) * LANES
                v = idxt_v[r, pl.ds(c0, LANES)]
                hi = v >= H
                pidx_v[r, pl.ds(c0, LANES)] = v - jnp.where(hi, H, 0)
                hv_v[r, pl.ds(c0, LANES)] = jnp.where(hi, MODEL_DIM, 0)

            gets = [None] * TT
            puts = [None] * TT
            gets[0] = pltpu.async_copy(pairs_hbm.at[pidx_v.at[0, :]],
                                       prows[0], gsem)
            for t8 in range(TT):
                cur = t8 & 1
                gets[t8].wait()
                if t8 + 1 < TT:
                    gets[t8 + 1] = pltpu.async_copy(
                        pairs_hbm.at[pidx_v.at[t8 + 1, :]],
                        prows[1 - cur], gsem)
                if t8 >= 2:
                    puts[t8 - 2].wait()
                src = prows[cur]
                dst = trans[cur]

                for gi in range(IB // LANES):
                    hv16 = hv_v[t8, pl.ds(gi * LANES, LANES)]
                    rowv = lane + gi * LANES

                    @plsc.parallel_loop(0, MODEL_DIM, unroll=8)
                    def col(c):
                        val = plsc.load_gather(src, [rowv, hv16 + c])
                        dst[c, pl.ds(gi * LANES, LANES)] = val * SCALE

                puts[t8] = pltpu.async_copy(
                    dst, out_hbm.at[tb * TT + t8, :, pl.ds(ib * IB, IB)],
                    wsem)
            puts[TT - 2].wait()
            puts[TT - 1].wait()
            return 0

        lax.fori_loop(0, units_w, unit, 0)

    return lkp


def kernel(x, table):
    T, N = x.shape[1], x.shape[0]
    xt = x.T.astype(jnp.int32)
    tt = table.T
    pairs = _make_pair_kernel()(tt, tt)
    out_t = _make_lookup_kernel(T, N)(xt, pairs)
    return jnp.transpose(out_t, (2, 0, 1))


# speedup vs baseline: 1.9968x; 1.0631x over previous
import functools
import math

import jax
import jax.numpy as jnp
from jax import lax
from jax.experimental import pallas as pl
from jax.experimental.pallas import tpu as pltpu
from jax.experimental.pallas import tpu_sc as plsc

MODEL_DIM = 64
VOCAB = 1000000
N_PAIR = VOCAB // 2
SCALE = math.sqrt(MODEL_DIM)

NUM_CORES = 2
NUM_SUBCORES = 16
NUM_WORKERS = NUM_CORES * NUM_SUBCORES
LANES = 16

TT = 8
IB = 128

_MESH = dict(core_axis_name="c", subcore_axis_name="s",
             num_cores=NUM_CORES, num_subcores=NUM_SUBCORES)

TC_W = 4096
N_GRID = 123
H = N_GRID * TC_W


def _make_pair_kernel():
    def body(lo_ref, hi_ref, out_ref):
        out_ref[:, 0:MODEL_DIM] = lo_ref[...].T
        out_ref[:, MODEL_DIM:2 * MODEL_DIM] = hi_ref[...].T

    return pl.pallas_call(
        body,
        grid=(N_GRID,),
        in_specs=[
            pl.BlockSpec((MODEL_DIM, TC_W), lambda j: (0, j)),
            pl.BlockSpec((MODEL_DIM, TC_W),
                         lambda j: (0, jnp.minimum(j + N_GRID,
                                                   VOCAB // TC_W))),
        ],
        out_specs=pl.BlockSpec((TC_W, 2 * MODEL_DIM), lambda j: (j, 0)),
        out_shape=jax.ShapeDtypeStruct((H, 2 * MODEL_DIM), jnp.float32),
    )


def _make_lookup_kernel(T: int, N: int):
    n_ib = N // IB
    units = (T // TT) * n_ib
    units_w = units // NUM_WORKERS

    @functools.partial(
        pl.kernel,
        out_type=jax.ShapeDtypeStruct((T, MODEL_DIM, N), jnp.float32),
        mesh=plsc.VectorSubcoreMesh(**_MESH),
        scratch_types=[
            pltpu.VMEM((TT, IB), jnp.int32),
            pltpu.VMEM((TT, IB), jnp.int32),
            pltpu.VMEM((TT, IB), jnp.int32),
            pltpu.VMEM((IB, 2 * MODEL_DIM), jnp.float32),
            pltpu.VMEM((IB, 2 * MODEL_DIM), jnp.float32),
            pltpu.VMEM((MODEL_DIM, IB), jnp.float32),
            pltpu.VMEM((MODEL_DIM, IB), jnp.float32),
            pltpu.SemaphoreType.DMA,
            pltpu.SemaphoreType.DMA,
        ],
        compiler_params=pltpu.CompilerParams(use_tc_tiling_on_sc=True,
                                             needs_layout_passes=False),
    )
    def lkp(xt_hbm, pairs_hbm, out_hbm, idxt_v, pidx_v, hv_v,
            prows_a, prows_b, trans_a, trans_b, gsem, wsem):
        wid = lax.axis_index("s") * NUM_CORES + lax.axis_index("c")
        lane = lax.iota(jnp.int32, LANES)
        prows = (prows_a, prows_b)
        trans = (trans_a, trans_b)

        def unit(k, _):
            u = wid + NUM_WORKERS * k
            tb = u // n_ib
            ib = u % n_ib
            pltpu.sync_copy(
                xt_hbm.at[pl.ds(tb * TT, TT), pl.ds(ib * IB, IB)], idxt_v)

            @plsc.parallel_loop(0, TT * (IB // LANES), unroll=8)
            def mkpidx(q):
                r = q // (IB // LANES)
                c0 = (q % (IB // LANES)) * LANES
                v = idxt_v[r, pl.ds(c0, LANES)]
                hi = v >= H
                pidx_v[r, pl.ds(c0, LANES)] = v - jnp.where(hi, H, 0)
                hv_v[r, pl.ds(c0, LANES)] = jnp.where(hi, MODEL_DIM, 0)

            gets = [None] * TT
            puts = [None] * TT
            gets[0] = pltpu.async_copy(pairs_hbm.at[pidx_v.at[0, :]],
                                       prows[0], gsem)
            for t8 in range(TT):
                cur = t8 & 1
                gets[t8].wait()
                if t8 + 1 < TT:
                    gets[t8 + 1] = pltpu.async_copy(
                        pairs_hbm.at[pidx_v.at[t8 + 1, :]],
                        prows[1 - cur], gsem)
                if t8 >= 2:
                    puts[t8 - 2].wait()
                src = prows[cur]
                dst = trans[cur]

                for gi in range(IB // LANES):
                    hv16 = hv_v[t8, pl.ds(gi * LANES, LANES)]
                    rowv = lane + gi * LANES

                    @plsc.parallel_loop(0, MODEL_DIM, unroll=16)
                    def col(c):
                        val = plsc.load_gather(src, [rowv, hv16 + c])
                        dst[c, pl.ds(gi * LANES, LANES)] = val * SCALE

                puts[t8] = pltpu.async_copy(
                    dst, out_hbm.at[tb * TT + t8, :, pl.ds(ib * IB, IB)],
                    wsem)
            puts[TT - 2].wait()
            puts[TT - 1].wait()
            return 0

        lax.fori_loop(0, units_w, unit, 0)

    return lkp


def kernel(x, table):
    T, N = x.shape[1], x.shape[0]
    xt = x.T.astype(jnp.int32)
    tt = table.T
    pairs = _make_pair_kernel()(tt, tt)
    out_t = _make_lookup_kernel(T, N)(xt, pairs)
    return jnp.transpose(out_t, (2, 0, 1))
